# Initial kernel scaffold; baseline (speedup 1.0000x reference)
import jax
import jax.numpy as jnp
from jax.experimental import pallas as pl


def kernel(x, edge_index, W1, a1s, a1d, b1, W2, a2s, a2d, b2):
    raise NotImplementedError("write your pallas kernel here")


# speedup vs baseline: 211.5694x; 211.5694x over previous
import functools

import jax
import jax.numpy as jnp
from jax import lax
from jax.experimental import pallas as pl
from jax.experimental.pallas import tpu as pltpu
from jax.experimental.pallas import tpu_sc as plsc

N = 10000
NP = 10208
E = 320000
EP = 322560
NW = 32
EW = EP // NW
C = 80
NCH = EW // C
RPS = NP // 16
DUMMY = NP - 1
HW = 128
NH = 8
FH = 16


def _prep_tc(x, w1f, a1sf, a1df, smat):
    def body(x_ref, w_ref, s_ref, d_ref, sm_ref, hs_ref, adt_ref):
        h = jnp.dot(x_ref[...], w_ref[...], preferred_element_type=jnp.float32)
        asn = jnp.dot(h * s_ref[...], sm_ref[...],
                      preferred_element_type=jnp.float32)
        adn = jnp.dot(h * d_ref[...], sm_ref[...],
                      preferred_element_type=jnp.float32)
        hs_ref[...] = jnp.concatenate([h, asn, adn], axis=1)
        adt_ref[...] = jnp.concatenate(
            [adn, jnp.zeros((NP, NH), jnp.float32)], axis=1)

    return pl.pallas_call(
        body,
        out_shape=(jax.ShapeDtypeStruct((NP, HW + 2 * NH), jnp.float32),
                   jax.ShapeDtypeStruct((NP, 2 * NH), jnp.float32)),
    )(x, w1f, a1sf, a1df, smat)


NB = 3
G = NCH // NB
NCH0 = 159
NCH1 = 93
EW0 = NCH0 * C
EW1 = NCH1 * C


def _sc_layer1(hs, adt, zeros144, src, dst):
    mesh = plsc.VectorSubcoreMesh(core_axis_name="c", subcore_axis_name="s")

    scr = ([pltpu.VMEM((C,), jnp.int32)] * NB
           + [pltpu.VMEM((C,), jnp.int32)] * NB
           + [pltpu.VMEM((C,), jnp.int32)] * NB
           + [pltpu.VMEM((C, 144), jnp.float32)] * NB
           + [pltpu.VMEM((C, 16), jnp.float32)] * NB
           + [pltpu.VMEM_SHARED((NP, 144), jnp.float32)]
           + [pltpu.SemaphoreType.DMA] * (5 * NB))

    @functools.partial(
        pl.kernel,
        out_type=jax.ShapeDtypeStruct((2 * NP, 144), jnp.float32),
        mesh=mesh,
        compiler_params=pltpu.CompilerParams(use_tc_tiling_on_sc=False,
                                             needs_layout_passes=False),
        scratch_types=scr,
    )
    def k(hs_hbm, adt_hbm, z_hbm, src_hbm, dst_hbm, out_hbm, *s):
        sidx = s[0:NB]
        didx = s[NB:2 * NB]
        sdidx = s[2 * NB:3 * NB]
        rows = s[3 * NB:4 * NB]
        adrows = s[4 * NB:5 * NB]
        acc_sh = s[5 * NB]
        isem_s = s[5 * NB + 1:5 * NB + 1 + NB]
        isem_d = s[5 * NB + 1 + NB:5 * NB + 1 + 2 * NB]
        gsem_h = s[5 * NB + 1 + 2 * NB:5 * NB + 1 + 3 * NB]
        gsem_a = s[5 * NB + 1 + 3 * NB:5 * NB + 1 + 4 * NB]
        ssem = s[5 * NB + 1 + 4 * NB:5 * NB + 1 + 5 * NB]

        cid = lax.axis_index("c")
        sid = lax.axis_index("s")
        r0 = sid * RPS
        ew_c = jnp.where(cid == 0, EW0, EW1)
        g_c = jnp.where(cid == 0, NCH0 // NB, NCH1 // NB)
        ebase = cid * 16 * EW0 + sid * ew_c

        def idx_start(kk, j):
            b = ebase + kk * C
            pltpu.async_copy(src_hbm.at[pl.ds(b, C)], sidx[j], isem_s[j])
            pltpu.async_copy(dst_hbm.at[pl.ds(b, C)], didx[j], isem_d[j])

        def idx_wait(j):
            pltpu.make_async_copy(src_hbm.at[pl.ds(0, C)], sidx[j], isem_s[j]).wait()
            pltpu.make_async_copy(dst_hbm.at[pl.ds(0, C)], didx[j], isem_d[j]).wait()

        def gat_start(j):
            pltpu.async_copy(hs_hbm.at[sidx[j]], rows[j], gsem_h[j])
            pltpu.async_copy(adt_hbm.at[didx[j]], adrows[j], gsem_a[j])

        def gat_wait(j):
            pltpu.make_async_copy(hs_hbm.at[sidx[j]], rows[j], gsem_h[j]).wait()
            pltpu.make_async_copy(adt_hbm.at[didx[j]], adrows[j], gsem_a[j]).wait()

        def scat_start(j):
            pltpu.async_copy(rows[j], acc_sh.at[sdidx[j]], ssem[j], add=True)

        def scat_wait(j):
            pltpu.make_async_copy(rows[j], acc_sh.at[sdidx[j]], ssem[j]).wait()

        for j in range(NB):
            idx_start(j, j)
        pltpu.sync_copy(z_hbm.at[pl.ds(r0, RPS), :], acc_sh.at[pl.ds(r0, RPS), :])
        plsc.subcore_barrier()

        def giter(g, carry):
            for j in range(NB):
                @pl.when(g > 0)
                def _(j=j):
                    scat_wait(j)
                idx_wait(j)
                gat_start(j)
            for j in range(NB):
                gat_wait(j)
                for t in range(C // 16):
                    sdidx[j][pl.ds(t * 16, 16)] = didx[j][pl.ds(t * 16, 16)]

                @pl.when(g < g_c - 1)
                def _(g=g, j=j):
                    idx_start(NB * g + NB + j, j)

                @plsc.parallel_loop(0, C, unroll=4)
                def edge(e, j=j):
                    asv = rows[j][e, pl.ds(HW, 16)]
                    adv = adrows[j][e, :]
                    ev = asv + adv
                    ev = jnp.maximum(ev, 0.2 * ev)
                    wv = jnp.exp(ev)
                    rows[j][e, pl.ds(HW, 16)] = wv
                    for hd in range(NH):
                        ws = wv[hd]
                        rows[j][e, pl.ds(hd * FH, FH)] = (
                            rows[j][e, pl.ds(hd * FH, FH)] * ws)
                scat_start(j)
            return carry

        lax.fori_loop(0, g_c, giter, 0)
        for j in range(NB):
            scat_wait(j)
        plsc.subcore_barrier()
        pltpu.sync_copy(acc_sh.at[pl.ds(r0, RPS), :],
                        out_hbm.at[pl.ds(cid * NP + r0, RPS), :])

    return k(hs, adt, zeros144, src, dst)


def _combine_tc(acc, hs, b1f, w2, a2sv, a2dv, stt):
    def body(acc_ref, hs_ref, b1_ref, w2_ref, s_ref, d_ref, st_ref,
             h2_ref, as2_ref, ad2_ref):
        accsum = acc_ref[0:NP, :] + acc_ref[NP:2 * NP, :]
        asn = hs_ref[:, HW:HW + NH]
        adn = hs_ref[:, HW + NH:HW + 2 * NH]
        ev = asn + adn
        wself = jnp.exp(jnp.maximum(ev, 0.2 * ev))
        wwide = jnp.dot(wself, st_ref[...],
                        preferred_element_type=jnp.float32)
        den8 = accsum[:, HW:HW + NH] + wself
        dwide = jnp.dot(den8, st_ref[...],
                        preferred_element_type=jnp.float32)
        num = accsum[:, 0:HW] + wwide * hs_ref[:, 0:HW]
        xh = num / (dwide + 1e-16) + b1_ref[...]
        h2 = jnp.dot(xh, w2_ref[...], preferred_element_type=jnp.float32)
        h2_ref[...] = h2
        as2_ref[...] = h2 * s_ref[...]
        ad2_ref[...] = h2 * d_ref[...]

    return pl.pallas_call(
        body,
        out_shape=(jax.ShapeDtypeStruct((NP, 1), jnp.float32),
                   jax.ShapeDtypeStruct((NP, 1), jnp.float32),
                   jax.ShapeDtypeStruct((NP, 1), jnp.float32)),
    )(acc, hs, b1f, w2, a2sv, a2dv, stt)


def _sc_layer2(h2a, as2a, ad2a, zeros16, src, dst):
    mesh = plsc.VectorSubcoreMesh(core_axis_name="c", subcore_axis_name="s")

    NB2 = 6
    G2 = NCH // NB2
    scr = ([pltpu.VMEM((C,), jnp.int32)] * NB2
           + [pltpu.VMEM((C,), jnp.int32)] * NB2
           + [pltpu.VMEM((C,), jnp.int32)] * NB2
           + [pltpu.VMEM((C,), jnp.int32)] * NB2
           + [pltpu.VMEM((C, 16), jnp.float32)] * NB2
           + [pltpu.VMEM((NP,), jnp.float32)] * 3
           + [pltpu.VMEM_SHARED((NP, 16), jnp.float32)]
           + [pltpu.SemaphoreType.DMA] * (3 * NB2))

    @functools.partial(
        pl.kernel,
        out_type=jax.ShapeDtypeStruct((2 * NP, 16), jnp.float32),
        mesh=mesh,
        compiler_params=pltpu.CompilerParams(use_tc_tiling_on_sc=False,
                                             needs_layout_passes=False),
        scratch_types=scr,
    )
    def k(h2_hbm, as2_hbm, ad2_hbm, z_hbm, src_hbm, dst_hbm, out_hbm, *s):
        sidx = s[0:NB2]
        didx = s[NB2:2 * NB2]
        ssidx = s[2 * NB2:3 * NB2]
        sdidx = s[3 * NB2:4 * NB2]
        crows = s[4 * NB2:5 * NB2]
        h2v, as2v, ad2v = s[5 * NB2:5 * NB2 + 3]
        acc_sh = s[5 * NB2 + 3]
        isem_s = s[5 * NB2 + 4:5 * NB2 + 4 + NB2]
        isem_d = s[5 * NB2 + 4 + NB2:5 * NB2 + 4 + 2 * NB2]
        ssem = s[5 * NB2 + 4 + 2 * NB2:5 * NB2 + 4 + 3 * NB2]

        cid = lax.axis_index("c")
        sid = lax.axis_index("s")
        wid = cid * 16 + sid
        r0 = sid * RPS
        pltpu.sync_copy(z_hbm.at[pl.ds(r0, RPS), :], acc_sh.at[pl.ds(r0, RPS), :])
        pltpu.sync_copy(h2_hbm, h2v)
        pltpu.sync_copy(as2_hbm, as2v)
        pltpu.sync_copy(ad2_hbm, ad2v)
        ebase = wid * EW

        def zrow(r, c2):
            for j in range(NB2):
                crows[j][r, :] = jnp.zeros((16,), jnp.float32)
            return c2
        lax.fori_loop(0, C, zrow, 0)
        plsc.subcore_barrier()

        def idx_start(kk, j):
            b = ebase + kk * C
            pltpu.async_copy(src_hbm.at[pl.ds(b, C)], sidx[j], isem_s[j])
            pltpu.async_copy(dst_hbm.at[pl.ds(b, C)], didx[j], isem_d[j])

        def idx_wait(j):
            pltpu.make_async_copy(src_hbm.at[pl.ds(0, C)], sidx[j], isem_s[j]).wait()
            pltpu.make_async_copy(dst_hbm.at[pl.ds(0, C)], didx[j], isem_d[j]).wait()

        def scat_start(j):
            pltpu.async_copy(crows[j], acc_sh.at[sdidx[j]], ssem[j], add=True)

        def scat_wait(j):
            pltpu.make_async_copy(crows[j], acc_sh.at[sdidx[j]], ssem[j]).wait()

        lane = lax.iota(jnp.int32, 16)
        zcol = jnp.zeros((16,), jnp.int32)
        onecol = zcol + 1

        for j in range(NB2):
            idx_start(j, j)

        def giter(g, carry):
            for j in range(NB2):
                idx_wait(j)
                for t in range(C // 16):
                    ssidx[j][pl.ds(t * 16, 16)] = sidx[j][pl.ds(t * 16, 16)]
                    sdidx[j][pl.ds(t * 16, 16)] = didx[j][pl.ds(t * 16, 16)]

                @pl.when(g < G2 - 1)
                def _(g=g, j=j):
                    idx_start(NB2 * g + NB2 + j, j)

                @pl.when(g > 0)
                def _(j=j):
                    scat_wait(j)

                @plsc.parallel_loop(0, C // 16, unroll=5)
                def grp(t, j=j):
                    sv = ssidx[j][pl.ds(t * 16, 16)]
                    dv = sdidx[j][pl.ds(t * 16, 16)]
                    av = plsc.load_gather(as2v, [sv])
                    bv = plsc.load_gather(ad2v, [dv])
                    hv = plsc.load_gather(h2v, [sv])
                    ev = av + bv
                    wv = jnp.exp(jnp.maximum(ev, 0.2 * ev))
                    rowv = lane + t * 16
                    plsc.store_scatter(crows[j], [rowv, zcol], wv * hv)
                    plsc.store_scatter(crows[j], [rowv, onecol], wv)
                scat_start(j)
            return carry

        lax.fori_loop(0, G2, giter, 0)
        for j in range(NB2):
            scat_wait(j)
        plsc.subcore_barrier()
        pltpu.sync_copy(acc_sh.at[pl.ds(r0, RPS), :],
                        out_hbm.at[pl.ds(cid * NP + r0, RPS), :])

    return k(h2a, as2a, ad2a, zeros16, src, dst)


def _final_tc(acc2, h2a, as2a, ad2a, b2v):
    def body(a_ref, h2_ref, s_ref, d_ref, b_ref, o_ref):
        num = a_ref[0:NP, 0:1] + a_ref[NP:2 * NP, 0:1]
        den = a_ref[0:NP, 1:2] + a_ref[NP:2 * NP, 1:2]
        ev = s_ref[...] + d_ref[...]
        wself = jnp.exp(jnp.maximum(ev, 0.2 * ev))
        num = num + wself * h2_ref[...]
        den = den + wself
        v = num / (den + 1e-16) + b_ref[...]
        o_ref[...] = jnp.where(v > 0, v, jnp.exp(jnp.minimum(v, 0.0)) - 1.0)

    return pl.pallas_call(
        body,
        out_shape=jax.ShapeDtypeStruct((NP, 1), jnp.float32),
    )(acc2, h2a, as2a, ad2a, b2v)


def kernel(x, edge_index, W1, a1s, a1d, b1, W2, a2s, a2d, b2):
    f32 = jnp.float32
    xp = jnp.pad(x.astype(f32), ((0, NP - N), (0, 0)))
    w1f = W1.astype(f32).transpose(1, 0, 2).reshape(x.shape[1], HW)
    a1sf = a1s.astype(f32).reshape(1, HW)
    a1df = a1d.astype(f32).reshape(1, HW)
    b1f = b1.astype(f32).reshape(1, HW)
    heads = (jnp.arange(HW, dtype=jnp.int32) // FH)
    smat = (heads[:, None] == jnp.arange(NH, dtype=jnp.int32)[None, :]).astype(f32)
    stt = smat.T

    src = jnp.pad(edge_index[0].astype(jnp.int32), (0, EP - E),
                  constant_values=DUMMY)
    dst = jnp.pad(edge_index[1].astype(jnp.int32), (0, EP - E),
                  constant_values=DUMMY)

    hs, adt = _prep_tc(xp, w1f, a1sf, a1df, smat)
    z144 = jnp.zeros((NP, 144), f32)
    acc = _sc_layer1(hs, adt, z144, src, dst)

    h2a, as2a, ad2a = _combine_tc(acc, hs, b1f, W2.astype(f32),
                                  a2s.astype(f32).reshape(1, 1),
                                  a2d.astype(f32).reshape(1, 1), stt)

    z16 = jnp.zeros((NP, 16), f32)
    acc2 = _sc_layer2(h2a.reshape(NP), as2a.reshape(NP), ad2a.reshape(NP),
                      z16, src, dst)

    out = _final_tc(acc2, h2a, as2a, ad2a, b2.astype(f32).reshape(1, 1))
    return out[:N, :]
